# baseline (speedup 1.0000x reference)
import functools

import jax
import jax.numpy as jnp
from jax import lax
from jax.experimental import pallas as pl
from jax.experimental.pallas import tpu as pltpu
from jax.experimental.pallas import tpu_sc as plsc

_NC = 2
_NS = 16
_NW = _NC * _NS
_NSLOT = 4
_R = 8


def _sc_gather_add(x2d, idx, table):
    V, D = table.shape
    N = idx.shape[0]
    n_per_w = N // _NW
    R = _R
    n_chunks = n_per_w // R
    mesh = plsc.VectorSubcoreMesh(core_axis_name="c", subcore_axis_name="s")

    @functools.partial(
        pl.kernel, mesh=mesh,
        out_type=jax.ShapeDtypeStruct((N, D), jnp.float32),
        scratch_types=[
            pltpu.VMEM((n_per_w,), jnp.int32),
            pltpu.VMEM((_NSLOT, R, D), jnp.float32),
            pltpu.VMEM((_NSLOT, R, D), jnp.float32),
            pltpu.SemaphoreType.DMA((_NSLOT,)),
            pltpu.SemaphoreType.DMA((_NSLOT,)),
            pltpu.SemaphoreType.DMA((_NSLOT,)),
        ],
    )
    def k(x_hbm, idx_hbm, table_hbm, out_hbm, idx_v, o_v, pe_v,
          xsem, gsem, osem):
        wid = lax.axis_index("s") * _NC + lax.axis_index("c")
        base = wid * n_per_w
        pltpu.sync_copy(idx_hbm.at[pl.ds(base, n_per_w)], idx_v)

        def start_fetch(c, b):
            pltpu.async_copy(
                x_hbm.at[pl.ds(base + c * R, R)], o_v.at[b], xsem.at[b])
            pltpu.async_copy(
                table_hbm.at[idx_v.at[pl.ds(c * R, R)]], pe_v.at[b],
                gsem.at[b])

        def wait_fetch(b):
            pltpu.make_async_copy(
                x_hbm.at[pl.ds(0, R)], o_v.at[b], xsem.at[b]).wait()
            pltpu.make_async_copy(
                table_hbm.at[pl.ds(0, R)], pe_v.at[b], gsem.at[b]).wait()

        def wait_out(b):
            pltpu.make_async_copy(
                o_v.at[b], out_hbm.at[pl.ds(0, R)], osem.at[b]).wait()

        start_fetch(0, 0)
        start_fetch(1, 1)

        @pl.loop(0, n_chunks, step=_NSLOT)
        def _(c):
            for b in range(_NSLOT):
                cc = c + b
                wait_fetch(b)

                @pl.loop(0, R)
                def _(r):
                    @pl.loop(0, D, step=64)
                    def _(col):
                        for u in range(4):
                            s = pl.ds(col + u * 16, 16)
                            plsc.addupdate(
                                o_v.at[b, r, s], pe_v.at[b, r, s][...])

                pltpu.async_copy(
                    o_v.at[b], out_hbm.at[pl.ds(base + cc * R, R)],
                    osem.at[b])

                b2 = (b + 2) % _NSLOT

                @pl.when(cc + 2 < n_chunks)
                def _():
                    @pl.when(cc >= 2)
                    def _():
                        wait_out(b2)
                    start_fetch(cc + 2, b2)

        for b in range(_NSLOT):
            wait_out(b)

    return k(x2d, idx, table)


def kernel(x, position_indices, pe_table):
    B, S, D = x.shape
    idx = position_indices.reshape(-1).astype(jnp.int32)
    out2d = _sc_gather_add(x.reshape(B * S, D), idx, pe_table)
    return out2d.reshape(B, S, D)
